# baseline (speedup 1.0000x reference)
import jax
import jax.numpy as jnp
from jax import lax
from jax.experimental import pallas as pl
from jax.experimental.pallas import tpu as pltpu
from jax.experimental.pallas import tpu_sc as plsc

D0, D1, D2 = 4096, 4096, 64
R = 16
BATCH = 524288

NC, NS = 2, 16
NW = NC * NS
PER_W = BATCH // NW
CH = 2048
NCH = PER_W // CH


def _body(ti_hbm, u0p_hbm, u1p_hbm, u2pr_hbm, out_hbm,
          u0p_v, u1p_v, u2pr_v, t_v, o_v, s_in):
    wid = lax.axis_index("s") * NC + lax.axis_index("c")
    base = wid * PER_W
    lanes = lax.iota(jnp.int32, 16)

    ins = [
        (u0p_hbm, u0p_v),
        (u1p_hbm, u1p_v),
        (u2pr_hbm, u2pr_v),
        (ti_hbm.at[pl.ds(base, PER_W)], t_v),
    ]
    for src, dst in ins:
        pltpu.make_async_copy(src, dst, s_in).start()
    for src, dst in ins:
        pltpu.make_async_copy(src, dst, s_in).wait()

    @plsc.parallel_loop(0, PER_W // 16, unroll=8)
    def grp(i):
        t = t_v[pl.ds(i * 16, 16)]
        a0 = t >> 18
        a1 = (t >> 6) & 4095
        a2 = ((t & 63) << 4) + lanes
        accs = [jnp.zeros((16,), jnp.float32) for _ in range(4)]
        for p in range(R // 2):
            v0 = plsc.load_gather(u0p_v.at[p], [a0])
            v1 = plsc.load_gather(u1p_v.at[p], [a1])
            v2 = plsc.load_gather(u2pr_v.at[p], [a2])
            w = plsc.bitcast(
                plsc.bitcast(v0, jnp.bfloat16)
                * plsc.bitcast(v1, jnp.bfloat16)
                * plsc.bitcast(v2, jnp.bfloat16),
                jnp.int32)
            pa = plsc.bitcast(w << 16, jnp.float32)
            pb = plsc.bitcast(w & -65536, jnp.float32)
            accs[(2 * p) % 4] = accs[(2 * p) % 4] + pa
            accs[(2 * p + 1) % 4] = accs[(2 * p + 1) % 4] + pb
        o_v[pl.ds(i * 16, 16)] = (accs[0] + accs[1]) + (accs[2] + accs[3])

    pltpu.sync_copy(o_v, out_hbm.at[pl.ds(base, PER_W)])


@jax.jit
def kernel(target_indices, U0, U1, U2):
    def pack(U):
        h = lax.bitcast_convert_type(U.astype(jnp.bfloat16), jnp.uint16)
        return lax.bitcast_convert_type(
            h[:, 0::2].astype(jnp.uint32)
            | (h[:, 1::2].astype(jnp.uint32) << 16),
            jnp.int32).T

    u0p = pack(U0)
    u1p = pack(U1)
    u2pr = jnp.broadcast_to(
        pack(U2)[:, :, None], (R // 2, D2, 16)).reshape(R // 2, D2 * 16)

    mesh = plsc.VectorSubcoreMesh(core_axis_name="c", subcore_axis_name="s")
    f = pl.kernel(
        _body,
        out_type=jax.ShapeDtypeStruct((BATCH,), jnp.float32),
        mesh=mesh,
        scratch_types=[
            pltpu.VMEM((R // 2, D0), jnp.int32),
            pltpu.VMEM((R // 2, D1), jnp.int32),
            pltpu.VMEM((R // 2, D2 * 16), jnp.int32),
            pltpu.VMEM((PER_W,), jnp.int32),
            pltpu.VMEM((PER_W,), jnp.float32),
            pltpu.SemaphoreType.DMA,
        ],
        compiler_params=pltpu.CompilerParams(
            needs_layout_passes=False, use_tc_tiling_on_sc=False,
            disable_bounds_checks=True,
        ),
    )
    return f(target_indices, u0p, u1p, u2pr)


# speedup vs baseline: 1.2402x; 1.2402x over previous
import jax
import jax.numpy as jnp
from jax import lax
from jax.experimental import pallas as pl
from jax.experimental.pallas import tpu as pltpu
from jax.experimental.pallas import tpu_sc as plsc

D0, D1, D2 = 4096, 4096, 64
R = 16
BATCH = 524288

NC, NS = 2, 16
NW = NC * NS
PER_W = BATCH // NW
CH = 2048
NCH = PER_W // CH


def _body(ti_hbm, u0p_hbm, u1p_hbm, u2pr_hbm, out_hbm,
          u0p_v, u1p_v, u2pr_v, t_v, o_v, s_in):
    wid = lax.axis_index("s") * NC + lax.axis_index("c")
    base = wid * PER_W
    lanes = lax.iota(jnp.int32, 16)

    ins = [
        (u0p_hbm, u0p_v),
        (u1p_hbm, u1p_v),
        (u2pr_hbm, u2pr_v),
        (ti_hbm.at[pl.ds(base, PER_W)], t_v),
    ]
    for src, dst in ins:
        pltpu.make_async_copy(src, dst, s_in).start()
    for src, dst in ins:
        pltpu.make_async_copy(src, dst, s_in).wait()

    @plsc.parallel_loop(0, PER_W // 16, unroll=4)
    def grp(i):
        t = t_v[pl.ds(i * 16, 16)]
        a0 = t >> 18
        a1 = (t >> 6) & 4095
        a2 = ((t & 63) << 4) + lanes
        accs = [jnp.zeros((16,), jnp.float32) for _ in range(4)]
        for p in range(R // 2):
            v0 = plsc.load_gather(u0p_v.at[p], [a0])
            v1 = plsc.load_gather(u1p_v.at[p], [a1])
            v2 = plsc.load_gather(u2pr_v.at[p], [a2])
            w = plsc.bitcast(
                plsc.bitcast(v0, jnp.bfloat16)
                * plsc.bitcast(v1, jnp.bfloat16)
                * plsc.bitcast(v2, jnp.bfloat16),
                jnp.int32)
            pa = plsc.bitcast(w << 16, jnp.float32)
            pb = plsc.bitcast(w & -65536, jnp.float32)
            accs[(2 * p) % 4] = accs[(2 * p) % 4] + pa
            accs[(2 * p + 1) % 4] = accs[(2 * p + 1) % 4] + pb
        o_v[pl.ds(i * 16, 16)] = (accs[0] + accs[1]) + (accs[2] + accs[3])

    pltpu.sync_copy(o_v, out_hbm.at[pl.ds(base, PER_W)])


@jax.jit
def kernel(target_indices, U0, U1, U2):
    def pack(U):
        h = lax.bitcast_convert_type(U.astype(jnp.bfloat16), jnp.uint16)
        return lax.bitcast_convert_type(
            h[:, 0::2].astype(jnp.uint32)
            | (h[:, 1::2].astype(jnp.uint32) << 16),
            jnp.int32).T

    u0p = pack(U0)
    u1p = pack(U1)
    u2pr = jnp.broadcast_to(
        pack(U2)[:, :, None], (R // 2, D2, 16)).reshape(R // 2, D2 * 16)

    mesh = plsc.VectorSubcoreMesh(core_axis_name="c", subcore_axis_name="s")
    f = pl.kernel(
        _body,
        out_type=jax.ShapeDtypeStruct((BATCH,), jnp.float32),
        mesh=mesh,
        scratch_types=[
            pltpu.VMEM((R // 2, D0), jnp.int32),
            pltpu.VMEM((R // 2, D1), jnp.int32),
            pltpu.VMEM((R // 2, D2 * 16), jnp.int32),
            pltpu.VMEM((PER_W,), jnp.int32),
            pltpu.VMEM((PER_W,), jnp.float32),
            pltpu.SemaphoreType.DMA,
        ],
        compiler_params=pltpu.CompilerParams(
            needs_layout_passes=False, use_tc_tiling_on_sc=False,
            disable_bounds_checks=True,
        ),
    )
    return f(target_indices, u0p, u1p, u2pr)
